# baseline (speedup 1.0000x reference)
import jax
import jax.numpy as jnp
from jax.experimental import pallas as pl
from jax.experimental.pallas import tpu as pltpu

_N, _D, _K = 16384, 256, 1024
_BM = 2048
_NS = 2


def _layer(r, r2, c2, s_ref, oh_ref):
    rc = jax.lax.dot_general(
        r.astype(jnp.bfloat16), s_ref[0][:, :_D], (((1,), (1,)), ((), ())),
        preferred_element_type=jnp.float32)
    dist2 = (r2 + c2) - 2.0 * rc
    m = jnp.min(dist2, axis=1, keepdims=True)
    ohm = jnp.where(dist2 == m, jnp.float32(1.0), jnp.float32(0.0))
    oh_ref[...] = ohm.astype(jnp.bfloat16)
    cnt = jnp.sum(ohm, axis=1, keepdims=True)

    @pl.when(jnp.any(cnt != jnp.float32(1.0)))
    def _():
        iota_f = jax.lax.broadcasted_iota(
            jnp.int32, dist2.shape, 1).astype(jnp.float32)
        idxf = jnp.min(jnp.where(dist2 <= m, iota_f, jnp.float32(_K)),
                       axis=1, keepdims=True)
        oh_ref[...] = (iota_f == idxf).astype(jnp.bfloat16)

    parts = jax.lax.dot_general(
        oh_ref[...], s_ref[1], (((1,), (0,)), ((), ())),
        preferred_element_type=jnp.float32)
    return ((parts[:, 0 * _D:1 * _D] + parts[:, 1 * _D:2 * _D])
            + parts[:, 2 * _D:3 * _D])


def _rq_body(x_ref, c0_ref, c1_ref, c2_ref, s0_ref, s1_ref, s2_ref,
             out_ref, n2_ref, oh_ref):
    @pl.when(pl.program_id(0) == 0)
    def _():
        for i, c_ref in enumerate((c0_ref, c1_ref, c2_ref)):
            c = c_ref[...]
            n2_ref[i, :] = jnp.sum(c * c, axis=1)

    sm = _BM // _NS
    rs = [x_ref[pl.ds(s * sm, sm), :] for s in range(_NS)]
    recons = [jnp.zeros_like(r) for r in rs]
    r2s = [jnp.sum(r * r, axis=1, keepdims=True) for r in rs]
    for li, s_ref in enumerate((s0_ref, s1_ref, s2_ref)):
        c2 = n2_ref[li, :][None, :]
        for s in range(_NS):
            q = _layer(rs[s], r2s[s], c2, s_ref, oh_ref.at[s])
            recons[s] = recons[s] + q
            rs[s] = rs[s] - q
            r2s[s] = jnp.sum(rs[s] * rs[s], axis=1, keepdims=True)
    for s in range(_NS):
        out_ref[pl.ds(s * sm, sm), :] = recons[s]


def _split_pack(c):
    t0 = jax.lax.optimization_barrier(c.astype(jnp.bfloat16))
    d1 = c - t0.astype(jnp.float32)
    t1 = jax.lax.optimization_barrier(d1.astype(jnp.bfloat16))
    t2 = (d1 - t1.astype(jnp.float32)).astype(jnp.bfloat16)
    split = jnp.concatenate([t0, t1, t2], axis=1)
    dist_op = jnp.concatenate([t0, t0, t0], axis=1)
    return jnp.stack([dist_op, split])


def _call(x, c0, c1, c2, s0, s1, s2, *, interpret=False):
    cspec = pl.BlockSpec((_K, _D), lambda i: (0, 0))
    sspec = pl.BlockSpec((2, _K, 3 * _D), lambda i: (0, 0, 0))
    return pl.pallas_call(
        _rq_body,
        grid=(_N // _BM,),
        in_specs=[pl.BlockSpec((_BM, _D), lambda i: (i, 0)),
                  cspec, cspec, cspec, sspec, sspec, sspec],
        out_specs=pl.BlockSpec((_BM, _D), lambda i: (i, 0)),
        out_shape=jax.ShapeDtypeStruct((_N, _D), jnp.float32),
        scratch_shapes=[pltpu.VMEM((8, _K), jnp.float32),
                        pltpu.VMEM((_NS, _BM // _NS, _K), jnp.bfloat16)],
        interpret=interpret,
    )(x, c0, c1, c2, s0, s1, s2)


@jax.jit
def kernel(x, c0, c1, c2):
    return _call(x, c0, c1, c2,
                 _split_pack(c0), _split_pack(c1), _split_pack(c2))


# speedup vs baseline: 1.3735x; 1.3735x over previous
import jax
import jax.numpy as jnp
from jax.experimental import pallas as pl
from jax.experimental.pallas import tpu as pltpu

_N, _D, _K = 16384, 256, 1024
_BM = 2048
_NS = 2


def _layer(r, r2, c2, s_ref):
    rc = jax.lax.dot_general(
        r.astype(jnp.bfloat16), s_ref[0][:, :_D], (((1,), (1,)), ((), ())),
        preferred_element_type=jnp.float32)
    dist2 = (r2 + c2) - 2.0 * rc
    m = jnp.min(dist2, axis=1, keepdims=True)
    iota_f = jax.lax.broadcasted_iota(
        jnp.int32, dist2.shape, 1).astype(jnp.float32)
    idxf = jnp.min(jnp.where(dist2 == m, iota_f, jnp.float32(_K)),
                   axis=1, keepdims=True)
    oh = (iota_f == idxf).astype(jnp.bfloat16)
    parts = jax.lax.dot_general(
        oh, s_ref[1], (((1,), (0,)), ((), ())),
        preferred_element_type=jnp.float32)
    return ((parts[:, 0 * _D:1 * _D] + parts[:, 1 * _D:2 * _D])
            + parts[:, 2 * _D:3 * _D])


def _rq_body(x_ref, c0_ref, c1_ref, c2_ref, s0_ref, s1_ref, s2_ref,
             out_ref, n2_ref):
    @pl.when(pl.program_id(0) == 0)
    def _():
        for i, c_ref in enumerate((c0_ref, c1_ref, c2_ref)):
            c = c_ref[...]
            n2_ref[i, :] = jnp.sum(c * c, axis=1)

    sm = _BM // _NS
    rs = [x_ref[pl.ds(s * sm, sm), :] for s in range(_NS)]
    recons = [jnp.zeros_like(r) for r in rs]
    r2s = [jnp.sum(r * r, axis=1, keepdims=True) for r in rs]
    for li, s_ref in enumerate((s0_ref, s1_ref, s2_ref)):
        c2 = n2_ref[li, :][None, :]
        for s in range(_NS):
            q = _layer(rs[s], r2s[s], c2, s_ref)
            recons[s] = recons[s] + q
            rs[s] = rs[s] - q
            r2s[s] = jnp.sum(rs[s] * rs[s], axis=1, keepdims=True)
    for s in range(_NS):
        out_ref[pl.ds(s * sm, sm), :] = recons[s]


def _split_pack(c):
    t0 = jax.lax.optimization_barrier(c.astype(jnp.bfloat16))
    d1 = c - t0.astype(jnp.float32)
    t1 = jax.lax.optimization_barrier(d1.astype(jnp.bfloat16))
    t2 = (d1 - t1.astype(jnp.float32)).astype(jnp.bfloat16)
    split = jnp.concatenate([t0, t1, t2], axis=1)
    dist_op = jnp.concatenate([t0, t0, t0], axis=1)
    return jnp.stack([dist_op, split])


def _call(x, c0, c1, c2, s0, s1, s2, *, interpret=False):
    cspec = pl.BlockSpec((_K, _D), lambda i: (0, 0))
    sspec = pl.BlockSpec((2, _K, 3 * _D), lambda i: (0, 0, 0))
    return pl.pallas_call(
        _rq_body,
        grid=(_N // _BM,),
        in_specs=[pl.BlockSpec((_BM, _D), lambda i: (i, 0)),
                  cspec, cspec, cspec, sspec, sspec, sspec],
        out_specs=pl.BlockSpec((_BM, _D), lambda i: (i, 0)),
        out_shape=jax.ShapeDtypeStruct((_N, _D), jnp.float32),
        scratch_shapes=[pltpu.VMEM((8, _K), jnp.float32)],
        interpret=interpret,
    )(x, c0, c1, c2, s0, s1, s2)


@jax.jit
def kernel(x, c0, c1, c2):
    return _call(x, c0, c1, c2,
                 _split_pack(c0), _split_pack(c1), _split_pack(c2))
